# Initial kernel scaffold; baseline (speedup 1.0000x reference)
import jax
import jax.numpy as jnp
from jax.experimental import pallas as pl


def kernel(feats, labels, domain_ids):
    raise NotImplementedError("write your pallas kernel here")


# speedup vs baseline: 2.0410x; 2.0410x over previous
import jax
import jax.numpy as jnp
from jax import lax
from jax.experimental import pallas as pl
from jax.experimental.pallas import tpu as pltpu
from jax.experimental.pallas import tpu_sc as plsc

_C = 1000
_D = 256
_M = 4
_MOM = 0.9
_B = 16384
_NSEG = _M * _C
_CP = 1024
_NSEGP = _M * _CP

_NC = 2
_NS = 16
_L = 16
_NW = _NC * _NS
_RPT = _B // _NW
_CHUNK = 128
_NCHUNK = _RPT // _CHUNK
_SEG_PT = _NSEGP // _NS


def _sc_body(feats_hbm, labels_hbm, domains_hbm, out_sums, out_cnts,
             fbuf, idx2d, lab_v, dom_v, cnt_v, acc_sh):
    cid = lax.axis_index("c")
    sid = lax.axis_index("s")
    wid = sid * _NC + cid
    base = wid * _RPT

    zrow = jnp.zeros((_L,), jnp.float32)

    def _fill_row(i, carry):
        for k in range(_D // _L):
            fbuf[i, pl.ds(k * _L, _L)] = zrow
        cnt_v[pl.ds(i * _L, _L)] = zrow
        cnt_v[pl.ds((i + _CHUNK) * _L, _L)] = zrow
        return carry

    lax.fori_loop(0, _CHUNK, _fill_row, 0)

    r0 = sid * _SEG_PT
    pltpu.sync_copy(fbuf, acc_sh.at[pl.ds(r0, _CHUNK)])
    pltpu.sync_copy(fbuf, acc_sh.at[pl.ds(r0 + _CHUNK, _CHUNK)])

    pltpu.sync_copy(labels_hbm.at[pl.ds(base, _RPT)], lab_v)
    pltpu.sync_copy(domains_hbm.at[pl.ds(base, _RPT)], dom_v)
    onesv = zrow + 1.0
    for j in range(_NCHUNK):
        for i in range(_CHUNK // _L):
            off = j * _CHUNK + i * _L
            seg = dom_v[pl.ds(off, _L)] * _CP + lab_v[pl.ds(off, _L)]
            idx2d[j, pl.ds(i * _L, _L)] = seg
            plsc.addupdate_scatter(cnt_v, [seg], onesv)

    plsc.subcore_barrier()

    for j in range(_NCHUNK):
        pltpu.sync_copy(feats_hbm.at[pl.ds(base + j * _CHUNK, _CHUNK)], fbuf)
        pltpu.sync_copy(fbuf, acc_sh.at[idx2d.at[j]], add=True)

    plsc.subcore_barrier()

    pltpu.sync_copy(acc_sh.at[pl.ds(r0, _SEG_PT)], out_sums.at[cid, pl.ds(r0, _SEG_PT)])
    pltpu.sync_copy(cnt_v, out_cnts.at[wid])


import functools


@functools.cache
def _get_sc_segsum():
    return pl.kernel(
        _sc_body,
        out_type=(
            jax.ShapeDtypeStruct((_NC, _NSEGP, _D), jnp.float32),
            jax.ShapeDtypeStruct((_NW, _NSEGP), jnp.float32),
        ),
        mesh=plsc.VectorSubcoreMesh(core_axis_name="c", subcore_axis_name="s"),
        compiler_params=pltpu.CompilerParams(use_tc_tiling_on_sc=False, needs_layout_passes=False),
        scratch_types=[
            pltpu.VMEM((_CHUNK, _D), jnp.float32),
            pltpu.VMEM((_NCHUNK, _CHUNK), jnp.int32),
            pltpu.VMEM((_RPT,), jnp.int32),
            pltpu.VMEM((_RPT,), jnp.int32),
            pltpu.VMEM((_NSEGP,), jnp.float32),
            pltpu.VMEM_SHARED((_NSEGP, _D), jnp.float32),
        ],
    )


_BLK = 512


def _mm_body(dom_ref, x_ref, s2_ref):
    i = pl.program_id(0)

    @pl.when(i == 0)
    def _init():
        s2_ref[...] = jnp.zeros_like(s2_ref)

    x = x_ref[...]
    dom = dom_ref[...]
    for d in range(_M):
        zd = jnp.where(dom == d, x, 0.0)
        s2_ref[d] += lax.dot_general(
            zd, x, (((0,), (0,)), ((), ())), preferred_element_type=jnp.float32)


def _tc_moments(domain2d, feats):
    return pl.pallas_call(
        _mm_body,
        grid=(_B // _BLK,),
        in_specs=[
            pl.BlockSpec((_BLK, 1), lambda i: (i, 0)),
            pl.BlockSpec((_BLK, _D), lambda i: (i, 0)),
        ],
        out_specs=pl.BlockSpec((_M, _D, _D), lambda i: (0, 0, 0)),
        out_shape=jax.ShapeDtypeStruct((_M, _D, _D), jnp.float32),
        compiler_params=pltpu.CompilerParams(dimension_semantics=("arbitrary",)),
    )(domain2d, feats)


def _outer(v):
    return lax.dot_general(v, v, (((0,), (0,)), ((), ())),
                           preferred_element_type=jnp.float32)


def _ep_body(sums_ref, cnts_ref, s2_ref, out_ref):
    sums = sums_ref[0] + sums_ref[1]
    cnts = jnp.sum(cnts_ref[...], axis=0)

    csafe = jnp.maximum(cnts, 1.0)
    mu = sums / csafe[:, :, None]
    present = cnts > 0.0
    presf = present.astype(jnp.float32)

    anchors = (1.0 - _MOM) * mu * presf[:, :, None]
    ag = jnp.zeros((_CP, _D), jnp.float32)
    for d in range(_M):
        upd = _MOM * ag + (1.0 - _MOM) * mu[d]
        pd = presf[d][:, None]
        ag = pd * upd + (1.0 - pd) * ag
    per = jnp.mean((anchors - ag[None]) ** 2, axis=-1)
    nvalid = jnp.sum(presf)
    caa = jnp.where(nvalid > 0,
                    jnp.sum(per * presf) / jnp.maximum(nvalid, 1.0),
                    0.0)

    s2 = s2_ref[...]
    tot = jnp.sum(sums, axis=(0, 1)).reshape(1, _D)
    mu_g = tot / _B
    s2_tot = jnp.sum(s2, axis=0)
    cov = (s2_tot - _B * _outer(mu_g)) / (_B + 1e-6)
    rows = lax.broadcasted_iota(jnp.int32, (_D, _D), 0)
    cols = lax.broadcasted_iota(jnp.int32, (_D, _D), 1)
    eye = (rows == cols).astype(jnp.float32)
    g_mean = (1.0 - _MOM) * mu_g
    g_cov = _MOM * eye + (1.0 - _MOM) * cov

    loss = jnp.float32(0.0)
    nval = jnp.float32(0.0)
    for d in range(_M):
        cnt = jnp.sum(cnts[d])
        s_row = jnp.sum(sums[d], axis=0).reshape(1, _D)
        mu_d = s_row / jnp.maximum(cnt, 1.0)
        cov_d = (s2[d] - cnt * _outer(mu_d)) / (cnt + 1e-6)
        l_d = jnp.mean((mu_d - g_mean) ** 2) + jnp.mean((cov_d - g_cov) ** 2)
        has = (cnt > 0).astype(jnp.float32)
        loss = loss + has * l_d
        nval = nval + has
    stats = jnp.where(nval > 0, loss / jnp.maximum(nval, 1.0), 0.0)

    out_ref[...] = jnp.full((1, 1), caa + stats, jnp.float32)


def _tc_epilogue(sums_p, cnts_p, s2):
    return pl.pallas_call(
        _ep_body,
        out_shape=jax.ShapeDtypeStruct((1, 1), jnp.float32),
    )(sums_p, cnts_p, s2)


def kernel(feats, labels, domain_ids):
    sums_p, cnts_p = _get_sc_segsum()(feats, labels, domain_ids)
    s2 = _tc_moments(domain_ids.reshape(_B, 1), feats)
    loss = _tc_epilogue(sums_p.reshape(_NC, _M, _CP, _D),
                        cnts_p.reshape(_NW, _M, _CP),
                        s2)
    return loss.reshape(())
